# baseline (speedup 1.0000x reference)
import functools

import jax
import jax.numpy as jnp
from jax.experimental import pallas as pl
from jax.experimental.pallas import tpu as pltpu
from jax.experimental.pallas import tpu_sc as plsc

_NC = 2
_NS = 16
_NW = _NC * _NS
_L = 16


def _sc_gather4(user_ids, item_ids, U_gmf, I_gmf, U_mlp, I_mlp):
    B = user_ids.shape[0]
    D = U_gmf.shape[1]
    bpw = B // _NW
    ng = bpw // _L
    out = jax.ShapeDtypeStruct((B, D), jnp.float32)
    mesh = plsc.VectorSubcoreMesh(core_axis_name="c", subcore_axis_name="s")

    @functools.partial(
        pl.kernel,
        out_type=(out, out, out, out),
        mesh=mesh,
        compiler_params=pltpu.CompilerParams(needs_layout_passes=False),
        scratch_types=[
            pltpu.VMEM((bpw,), jnp.int32),
            pltpu.VMEM((bpw,), jnp.int32),
            pltpu.SemaphoreType.DMA,
        ],
    )
    def gather_kernel(uid_hbm, iid_hbm, ug_hbm, ig_hbm, um_hbm, im_hbm,
                      oug, oig, oum, oim, uidx_v, iidx_v, sem):
        wid = jax.lax.axis_index("s") * _NC + jax.lax.axis_index("c")
        base = wid * bpw
        pltpu.sync_copy(uid_hbm.at[pl.ds(base, bpw)], uidx_v)
        pltpu.sync_copy(iid_hbm.at[pl.ds(base, bpw)], iidx_v)
        lane = jax.lax.iota(jnp.int32, _L)

        @pl.loop(0, ng)
        def _(j):
            uvec = uidx_v[pl.ds(j * _L, _L)]
            ivec = iidx_v[pl.ds(j * _L, _L)]
            for l in range(_L):
                u = jnp.sum(jnp.where(lane == l, uvec, 0))
                v = jnp.sum(jnp.where(lane == l, ivec, 0))
                row = pl.ds(base + j * _L + l, 1)
                pltpu.async_copy(ug_hbm.at[pl.ds(u, 1)], oug.at[row], sem)
                pltpu.async_copy(ig_hbm.at[pl.ds(v, 1)], oig.at[row], sem)
                pltpu.async_copy(um_hbm.at[pl.ds(u, 1)], oum.at[row], sem)
                pltpu.async_copy(im_hbm.at[pl.ds(v, 1)], oim.at[row], sem)

            @pl.when(j > 0)
            def _():
                grp = pl.ds(0, _L)
                dgrp = pl.ds(base, _L)
                pltpu.make_async_copy(ug_hbm.at[grp], oug.at[dgrp], sem).wait()
                pltpu.make_async_copy(ig_hbm.at[grp], oig.at[dgrp], sem).wait()
                pltpu.make_async_copy(um_hbm.at[grp], oum.at[dgrp], sem).wait()
                pltpu.make_async_copy(im_hbm.at[grp], oim.at[dgrp], sem).wait()

        grp = pl.ds(0, _L)
        dgrp = pl.ds(base, _L)
        pltpu.make_async_copy(ug_hbm.at[grp], oug.at[dgrp], sem).wait()
        pltpu.make_async_copy(ig_hbm.at[grp], oig.at[dgrp], sem).wait()
        pltpu.make_async_copy(um_hbm.at[grp], oum.at[dgrp], sem).wait()
        pltpu.make_async_copy(im_hbm.at[grp], oim.at[dgrp], sem).wait()

    return gather_kernel(user_ids, item_ids, U_gmf, I_gmf, U_mlp, I_mlp)


def _mlp_body(ug, ig, um, im, w1u, w1i, b1, w2, b2, w3, b3, wpg, wph, bp, out):
    f32 = jnp.float32
    um_b = um[...].astype(jnp.bfloat16)
    im_b = im[...].astype(jnp.bfloat16)
    h1 = jnp.maximum(
        jnp.dot(um_b, w1u[...], preferred_element_type=f32)
        + jnp.dot(im_b, w1i[...], preferred_element_type=f32)
        + b1[...], 0.0)
    h2 = jnp.maximum(
        jnp.dot(h1.astype(jnp.bfloat16), w2[...], preferred_element_type=f32)
        + b2[...], 0.0)
    h3 = jnp.maximum(
        jnp.dot(h2.astype(jnp.bfloat16), w3[...], preferred_element_type=f32)
        + b3[...], 0.0)
    gmf = ug[...] * ig[...]
    pred = (jnp.sum(gmf * wpg[...], axis=1)
            + jnp.sum(h3 * wph[...], axis=1)
            + bp[...][0, 0])
    out[...] = jax.nn.sigmoid(pred)


def _tc_mlp(ug, ig, um, im, W1, b1, W2, b2, W3, b3, Wp, bp):
    B, D = ug.shape
    H1 = W1.shape[1]
    H2 = W2.shape[1]
    H3 = W3.shape[1]
    BS = 2048
    bf16 = jnp.bfloat16
    w1u = W1[:D].astype(bf16)
    w1i = W1[D:].astype(bf16)
    w2 = W2.astype(bf16)
    w3 = W3.astype(bf16)
    wpg = Wp[:D].reshape(1, D)
    wph = Wp[D:].reshape(1, D)
    b1r = b1.reshape(1, H1)
    b2r = b2.reshape(1, H2)
    b3r = b3.reshape(1, H3)
    bpr = bp.reshape(1, 1)

    emb_spec = pl.BlockSpec((BS, D), lambda i: (i, 0))

    def full(a):
        return pl.BlockSpec(a.shape, lambda i: tuple(0 for _ in a.shape))

    return pl.pallas_call(
        _mlp_body,
        grid=(B // BS,),
        in_specs=[emb_spec, emb_spec, emb_spec, emb_spec,
                  full(w1u), full(w1i), full(b1r), full(w2), full(b2r),
                  full(w3), full(b3r), full(wpg), full(wph), full(bpr)],
        out_specs=pl.BlockSpec((BS,), lambda i: (i,)),
        out_shape=jax.ShapeDtypeStruct((B,), jnp.float32),
    )(ug, ig, um, im, w1u, w1i, b1r, w2, b2r, w3, b3r, wpg, wph, bpr)


def kernel(user_ids, item_ids, U_gmf, I_gmf, U_mlp, I_mlp,
           W1, b1, W2, b2, W3, b3, Wp, bp):
    ug, ig, um, im = _sc_gather4(user_ids, item_ids, U_gmf, I_gmf, U_mlp, I_mlp)
    return _tc_mlp(ug, ig, um, im, W1, b1, W2, b2, W3, b3, Wp, bp)


# speedup vs baseline: 1.7553x; 1.7553x over previous
import functools

import jax
import jax.numpy as jnp
from jax.experimental import pallas as pl
from jax.experimental.pallas import tpu as pltpu
from jax.experimental.pallas import tpu_sc as plsc

_NC = 2
_NS = 16
_NW = _NC * _NS


def _sc_gather2(user_ids, item_ids, Ucat, Icat):
    B = user_ids.shape[0]
    W = Ucat.shape[1]
    bpw = B // _NW
    ch = bpw // 2
    out = jax.ShapeDtypeStruct((B, W), jnp.float32)
    mesh = plsc.VectorSubcoreMesh(core_axis_name="c", subcore_axis_name="s")

    @functools.partial(
        pl.kernel,
        out_type=(out, out),
        mesh=mesh,
        scratch_types=[
            pltpu.VMEM((bpw,), jnp.int32),
            pltpu.VMEM((bpw,), jnp.int32),
            pltpu.VMEM((ch, W), jnp.float32),
            pltpu.VMEM((ch, W), jnp.float32),
            pltpu.SemaphoreType.DMA,
            pltpu.SemaphoreType.DMA,
        ],
    )
    def gather_kernel(uid_hbm, iid_hbm, u_hbm, i_hbm, ou, oi,
                      uidx_v, iidx_v, buf0, buf1, gsem, ssem):
        wid = jax.lax.axis_index("s") * _NC + jax.lax.axis_index("c")
        base = wid * bpw
        pltpu.sync_copy(uid_hbm.at[pl.ds(base, bpw)], uidx_v)
        pltpu.sync_copy(iid_hbm.at[pl.ds(base, bpw)], iidx_v)

        srcs = (u_hbm, u_hbm, i_hbm, i_hbm)
        idxs = (uidx_v, uidx_v, iidx_v, iidx_v)
        outs = (ou, ou, oi, oi)
        offs = (0, ch, 0, ch)

        prev = None
        prev_store = None
        for k in range(4):
            buf = buf0 if k % 2 == 0 else buf1
            g = pltpu.async_copy(
                srcs[k].at[idxs[k].at[pl.ds(offs[k], ch)]], buf, gsem)
            if prev is not None:
                pk, pbuf = prev
                if prev_store is not None:
                    prev_store.wait()
                prev_store = pltpu.async_copy(
                    pbuf, outs[pk].at[pl.ds(base + offs[pk], ch)], ssem)
            g.wait()
            prev = (k, buf)
        pk, pbuf = prev
        if prev_store is not None:
            prev_store.wait()
        pltpu.sync_copy(pbuf, outs[pk].at[pl.ds(base + offs[pk], ch)])

    return gather_kernel(user_ids, item_ids, Ucat, Icat)


def _mlp_body(uc, ic, w1u, w1i, b1, w2, b2, w3, b3, wpg, wph, bp, out):
    f32 = jnp.float32
    D = w1u.shape[0]
    ucat = uc[...]
    icat = ic[...]
    um_b = ucat[:, D:].astype(jnp.bfloat16)
    im_b = icat[:, D:].astype(jnp.bfloat16)
    h1 = jnp.maximum(
        jnp.dot(um_b, w1u[...], preferred_element_type=f32)
        + jnp.dot(im_b, w1i[...], preferred_element_type=f32)
        + b1[...], 0.0)
    h2 = jnp.maximum(
        jnp.dot(h1.astype(jnp.bfloat16), w2[...], preferred_element_type=f32)
        + b2[...], 0.0)
    h3 = jnp.maximum(
        jnp.dot(h2.astype(jnp.bfloat16), w3[...], preferred_element_type=f32)
        + b3[...], 0.0)
    gmf = ucat[:, :D] * icat[:, :D]
    pred = (jnp.sum(gmf * wpg[...], axis=1)
            + jnp.sum(h3 * wph[...], axis=1)
            + bp[...][0, 0])
    out[...] = jax.nn.sigmoid(pred)


def _tc_mlp(ucat, icat, W1, b1, W2, b2, W3, b3, Wp, bp):
    B, W = ucat.shape
    D = W // 2
    H1 = W1.shape[1]
    H2 = W2.shape[1]
    H3 = W3.shape[1]
    BS = 2048
    bf16 = jnp.bfloat16
    w1u = W1[:D].astype(bf16)
    w1i = W1[D:].astype(bf16)
    w2 = W2.astype(bf16)
    w3 = W3.astype(bf16)
    wpg = Wp[:D].reshape(1, D)
    wph = Wp[D:].reshape(1, D)
    b1r = b1.reshape(1, H1)
    b2r = b2.reshape(1, H2)
    b3r = b3.reshape(1, H3)
    bpr = bp.reshape(1, 1)

    emb_spec = pl.BlockSpec((BS, W), lambda i: (i, 0))

    def full(a):
        return pl.BlockSpec(a.shape, lambda i: tuple(0 for _ in a.shape))

    return pl.pallas_call(
        _mlp_body,
        grid=(B // BS,),
        in_specs=[emb_spec, emb_spec,
                  full(w1u), full(w1i), full(b1r), full(w2), full(b2r),
                  full(w3), full(b3r), full(wpg), full(wph), full(bpr)],
        out_specs=pl.BlockSpec((BS,), lambda i: (i,)),
        out_shape=jax.ShapeDtypeStruct((B,), jnp.float32),
    )(ucat, icat, w1u, w1i, b1r, w2, b2r, w3, b3r, wpg, wph, bpr)


def kernel(user_ids, item_ids, U_gmf, I_gmf, U_mlp, I_mlp,
           W1, b1, W2, b2, W3, b3, Wp, bp):
    Ucat = jnp.concatenate([U_gmf, U_mlp], axis=1)
    Icat = jnp.concatenate([I_gmf, I_mlp], axis=1)
    ucat, icat = _sc_gather2(user_ids, item_ids, Ucat, Icat)
    return _tc_mlp(ucat, icat, W1, b1, W2, b2, W3, b3, Wp, bp)
